# baseline (speedup 1.0000x reference)
import jax
import jax.numpy as jnp
from jax import lax
from jax.experimental import pallas as pl
from jax.experimental.pallas import tpu as pltpu
from jax.experimental.pallas import tpu_sc as plsc

B = 16384
F = 26
FA = 16
FB = F - FA
V = 100000
D = 32
NUM = 13
ED = F * D
NT = 7
NTA = FA // 4
NTB = NT - NTA
EPS = 1e-5

NC = 2
NS = 16
NW = NC * NS
ROWS_W = B // NW
RB = 128
NRB = ROWS_W // RB
NBUF = 4


def _make_body(nf, nt):
    def body(tab3, xt, out3, idx_v, rows_v, gsem):
        wid = lax.axis_index("s") * NC + lax.axis_index("c")
        b_base = wid * ROWS_W
        pltpu.sync_copy(xt.at[:, pl.ds(b_base, ROWS_W)], idx_v)
        chunks_w = NRB * nf

        def outer(co, carry):
            c0 = co * NBUF
            for b in range(NBUF):
                c = c0 + b
                f = c % nf
                rb = c // nf
                pltpu.async_copy(
                    tab3.at[f].at[idx_v.at[f, pl.ds(rb * RB, RB)]],
                    rows_v.at[b], gsem)
            for b in range(NBUF):
                c = c0 + b
                f = c % nf
                rb = c // nf
                pltpu.make_async_copy(
                    tab3.at[f].at[idx_v.at[f, pl.ds(rb * RB, RB)]],
                    rows_v.at[b], gsem).wait()
                ct = f // 4
                k = f % 4
                pltpu.sync_copy(
                    rows_v.at[b],
                    out3.at[ct, pl.ds(b_base + rb * RB, RB),
                            pl.ds(32 * k, 32)])
            return carry

        lax.fori_loop(0, chunks_w // NBUF, outer, 0)

    return body


_SC_GATHER_CACHE = {}


def _sc_gather(nf, nt, tab3, xt):
    if nf not in _SC_GATHER_CACHE:
        _SC_GATHER_CACHE[nf] = pl.kernel(
            _make_body(nf, nt),
            out_type=jax.ShapeDtypeStruct((nt, B, 128), jnp.float32),
            mesh=plsc.VectorSubcoreMesh(core_axis_name="c", subcore_axis_name="s"),
            scratch_types=[
                pltpu.VMEM((nf, ROWS_W), jnp.int32),
                pltpu.VMEM((NBUF, RB, D), jnp.float32),
                pltpu.SemaphoreType.DMA,
            ],
            compiler_params=pltpu.CompilerParams(use_tc_tiling_on_sc=False),
        )
    return _SC_GATHER_CACHE[nf](tab3, xt)


BB = 1024
_INV_STD = (1.0 + EPS) ** -0.5


def _mlp_body(xa, xb, xn, w1a, w1b, w1n, b1, g1, be1, w2, b2, g2, be2, w3, b3,
              out):
    h = jnp.dot(xa[0], w1a[0], preferred_element_type=jnp.float32)
    for t in range(1, NTA):
        h = h + jnp.dot(xa[t], w1a[t], preferred_element_type=jnp.float32)
    for t in range(NTB - 1):
        h = h + jnp.dot(xb[t], w1b[t], preferred_element_type=jnp.float32)
    col = lax.broadcasted_iota(jnp.int32, (BB, 128), 1)
    x6 = jnp.where(col < 64, xb[NTB - 1], 0.0)
    h = h + jnp.dot(x6, w1b[NTB - 1], preferred_element_type=jnp.float32)
    h = h + jnp.dot(xn[...], w1n[...], preferred_element_type=jnp.float32)
    h = jnp.maximum(h + b1[...], 0.0)
    h = h * (g1[...] * _INV_STD) + be1[...]
    h = jnp.maximum(jnp.dot(h, w2[...], preferred_element_type=jnp.float32) + b2[...], 0.0)
    h = h * (g2[...] * _INV_STD) + be2[...]
    out[...] = jnp.dot(h, w3[...], preferred_element_type=jnp.float32) + b3[...]


_mlp = pl.pallas_call(
    _mlp_body,
    grid=(B // BB,),
    in_specs=[
        pl.BlockSpec((NTA, BB, 128), lambda i: (0, i, 0)),
        pl.BlockSpec((NTB, BB, 128), lambda i: (0, i, 0)),
        pl.BlockSpec((BB, NUM), lambda i: (i, 0)),
        pl.BlockSpec((NTA, 128, 128), lambda i: (0, 0, 0)),
        pl.BlockSpec((NTB, 128, 128), lambda i: (0, 0, 0)),
        pl.BlockSpec((NUM, 128), lambda i: (0, 0)),
        pl.BlockSpec((1, 128), lambda i: (0, 0)),
        pl.BlockSpec((1, 128), lambda i: (0, 0)),
        pl.BlockSpec((1, 128), lambda i: (0, 0)),
        pl.BlockSpec((128, 64), lambda i: (0, 0)),
        pl.BlockSpec((1, 64), lambda i: (0, 0)),
        pl.BlockSpec((1, 64), lambda i: (0, 0)),
        pl.BlockSpec((1, 64), lambda i: (0, 0)),
        pl.BlockSpec((64, 1), lambda i: (0, 0)),
        pl.BlockSpec((1, 1), lambda i: (0, 0)),
    ],
    out_specs=pl.BlockSpec((BB, 1), lambda i: (i, 0)),
    out_shape=jax.ShapeDtypeStruct((B, 1), jnp.float32),
)


def kernel(x_cat, x_num, tables, W1, b1, g1, be1, W2, b2, g2, be2, W3, b3):
    xt = x_cat.T
    xa = _sc_gather(FA, NTA, tables[:FA], xt[:FA])
    xb = _sc_gather(FB, NTB, tables[FA:], xt[FA:])

    w1a = W1[:FA * D].reshape(NTA, 128, 128)
    w1b = jnp.concatenate(
        [W1[FA * D:ED], jnp.zeros((NT * 128 - ED, 128), W1.dtype)]
    ).reshape(NTB, 128, 128)
    return _mlp(
        xa, xb, x_num, w1a, w1b, W1[ED:],
        b1.reshape(1, 128), g1.reshape(1, 128), be1.reshape(1, 128),
        W2, b2.reshape(1, 64), g2.reshape(1, 64), be2.reshape(1, 64),
        W3, b3.reshape(1, 1),
    )
